# Initial kernel scaffold; baseline (speedup 1.0000x reference)
import jax
import jax.numpy as jnp
from jax.experimental import pallas as pl


def kernel(x, edge_index, edge_weight, W1, b1, W2, b2):
    raise NotImplementedError("write your pallas kernel here")


# speedup vs baseline: 6.0385x; 6.0385x over previous
import functools

import jax
import jax.numpy as jnp
from jax import lax
from jax.experimental import pallas as pl
from jax.experimental.pallas import tpu as pltpu
from jax.experimental.pallas import tpu_sc as plsc

N = 10000
NP = 10240
E = 320000
D_IN = 128
D_H = 256
D_OUT = 128

NC, NS, LANES = 2, 16, 16
NW = NC * NS
CHUNK = 128
CHUNKS_PER_W = 80
EDGES_PER_W = CHUNK * CHUNKS_PER_W
EP = NW * EDGES_PER_W
ROWS_PER_SUB = NP // NS

_MESH = plsc.VectorSubcoreMesh(core_axis_name="c", subcore_axis_name="s",
                               num_cores=NC, num_subcores=NS)
_SC_PARAMS = pltpu.CompilerParams(needs_layout_passes=False)


def _deg_body(dst_hbm, ew_hbm, zeros1_hbm, out_hbm, dstv, ewv, deg_sh):
    c = lax.axis_index("c")
    s = lax.axis_index("s")
    wid = c * NS + s
    row0 = s * ROWS_PER_SUB
    pltpu.sync_copy(zeros1_hbm.at[pl.ds(row0, ROWS_PER_SUB)],
                    deg_sh.at[pl.ds(row0, ROWS_PER_SUB)])
    plsc.subcore_barrier()

    def chunk(i, carry):
        base = wid * EDGES_PER_W + i * CHUNK
        pltpu.sync_copy(dst_hbm.at[pl.ds(base, CHUNK)], dstv)
        pltpu.sync_copy(ew_hbm.at[pl.ds(base, CHUNK)], ewv)
        pltpu.sync_copy(ewv, deg_sh.at[dstv], add=True)
        return carry

    lax.fori_loop(0, CHUNKS_PER_W, chunk, 0)
    plsc.subcore_barrier()
    pltpu.sync_copy(deg_sh.at[pl.ds(row0, ROWS_PER_SUB)],
                    out_hbm.at[c, pl.ds(row0, ROWS_PER_SUB)])


_deg_kernel = functools.partial(
    pl.kernel,
    out_type=jax.ShapeDtypeStruct((NC, NP), jnp.float32),
    mesh=_MESH,
    compiler_params=_SC_PARAMS,
    scratch_types=[
        pltpu.VMEM((CHUNK,), jnp.int32),
        pltpu.VMEM((CHUNK,), jnp.float32),
        pltpu.MemorySpace.VMEM_SHARED((NP,), jnp.float32),
    ],
)(_deg_body)


def _agg_body(u_hbm, src_hbm, dst_hbm, ew_hbm, zeros2_hbm, out_hbm,
              srcv, dstv, ewv, rows, t_sh, sem):
    c = lax.axis_index("c")
    s = lax.axis_index("s")
    wid = c * NS + s
    row0 = s * ROWS_PER_SUB
    pltpu.sync_copy(zeros2_hbm.at[pl.ds(row0, ROWS_PER_SUB)],
                    t_sh.at[pl.ds(row0, ROWS_PER_SUB)])
    plsc.subcore_barrier()

    def chunk(i, carry):
        base = wid * EDGES_PER_W + i * CHUNK
        pltpu.sync_copy(src_hbm.at[pl.ds(base, CHUNK)], srcv)
        pltpu.sync_copy(dst_hbm.at[pl.ds(base, CHUNK)], dstv)
        pltpu.sync_copy(ew_hbm.at[pl.ds(base, CHUNK)], ewv)
        pltpu.async_copy(u_hbm.at[srcv], rows, sem).wait()

        def scale(e, carry2):
            bew = plsc.load_gather(
                ewv, [jnp.zeros((LANES,), jnp.int32) + e])
            for f in range(D_IN // LANES):
                sl = pl.ds(f * LANES, LANES)
                rows[e, sl] = rows[e, sl] * bew
            return carry2

        lax.fori_loop(0, CHUNK, scale, 0)
        pltpu.sync_copy(rows, t_sh.at[dstv], add=True)
        return carry

    lax.fori_loop(0, CHUNKS_PER_W, chunk, 0)
    plsc.subcore_barrier()
    pltpu.sync_copy(t_sh.at[pl.ds(row0, ROWS_PER_SUB)],
                    out_hbm.at[c, pl.ds(row0, ROWS_PER_SUB)])


_agg_kernel = functools.partial(
    pl.kernel,
    out_type=jax.ShapeDtypeStruct((NC, NP, D_IN), jnp.float32),
    mesh=_MESH,
    compiler_params=_SC_PARAMS,
    scratch_types=[
        pltpu.VMEM((CHUNK,), jnp.int32),
        pltpu.VMEM((CHUNK,), jnp.int32),
        pltpu.VMEM((CHUNK,), jnp.float32),
        pltpu.VMEM((CHUNK, D_IN), jnp.float32),
        pltpu.MemorySpace.VMEM_SHARED((NP, D_IN), jnp.float32),
        pltpu.SemaphoreType.DMA,
    ],
)(_agg_body)


_BLK = 1024
_GRID = NP // _BLK


def _tc1_body(d0_ref, d1_ref, x_ref, dis_ref, u1_ref):
    deg = d0_ref[...] + d1_ref[...] + 1.0
    dis = 1.0 / jnp.sqrt(deg)
    dis_ref[...] = dis
    u1_ref[...] = x_ref[...] * dis


def _tc1(d0, d1, xp):
    return pl.pallas_call(
        _tc1_body,
        grid=(_GRID,),
        in_specs=[
            pl.BlockSpec((_BLK, 1), lambda i: (i, 0)),
            pl.BlockSpec((_BLK, 1), lambda i: (i, 0)),
            pl.BlockSpec((_BLK, D_IN), lambda i: (i, 0)),
        ],
        out_specs=[
            pl.BlockSpec((_BLK, 1), lambda i: (i, 0)),
            pl.BlockSpec((_BLK, D_IN), lambda i: (i, 0)),
        ],
        out_shape=[
            jax.ShapeDtypeStruct((NP, 1), jnp.float32),
            jax.ShapeDtypeStruct((NP, D_IN), jnp.float32),
        ],
    )(d0, d1, xp)


def _tc2_body(t0_ref, t1_ref, u1_ref, dis_ref, w1_ref, b1_ref, w2_ref,
              u2_ref):
    g1 = (t0_ref[...] + t1_ref[...] + u1_ref[...]) * dis_ref[...]
    h = jnp.dot(g1, w1_ref[...], preferred_element_type=jnp.float32,
                precision=lax.Precision.HIGHEST) + b1_ref[...]
    h = jnp.maximum(h, 0.0)
    u2 = jnp.dot(h, w2_ref[...], preferred_element_type=jnp.float32,
                 precision=lax.Precision.HIGHEST)
    u2_ref[...] = u2 * dis_ref[...]


def _tc2(t0, t1, u1, dis, W1, b1, W2):
    return pl.pallas_call(
        _tc2_body,
        grid=(_GRID,),
        in_specs=[
            pl.BlockSpec((_BLK, D_IN), lambda i: (i, 0)),
            pl.BlockSpec((_BLK, D_IN), lambda i: (i, 0)),
            pl.BlockSpec((_BLK, D_IN), lambda i: (i, 0)),
            pl.BlockSpec((_BLK, 1), lambda i: (i, 0)),
            pl.BlockSpec((D_IN, D_H), lambda i: (0, 0)),
            pl.BlockSpec((1, D_H), lambda i: (0, 0)),
            pl.BlockSpec((D_H, D_OUT), lambda i: (0, 0)),
        ],
        out_specs=pl.BlockSpec((_BLK, D_OUT), lambda i: (i, 0)),
        out_shape=jax.ShapeDtypeStruct((NP, D_OUT), jnp.float32),
    )(t0, t1, u1, dis, W1, b1, W2)


def _tc3_body(t0_ref, t1_ref, u2_ref, dis_ref, b2_ref, z_ref):
    z_ref[...] = ((t0_ref[...] + t1_ref[...] + u2_ref[...]) * dis_ref[...]
                  + b2_ref[...])


def _tc3(t0, t1, u2, dis, b2):
    return pl.pallas_call(
        _tc3_body,
        grid=(_GRID,),
        in_specs=[
            pl.BlockSpec((_BLK, D_OUT), lambda i: (i, 0)),
            pl.BlockSpec((_BLK, D_OUT), lambda i: (i, 0)),
            pl.BlockSpec((_BLK, D_OUT), lambda i: (i, 0)),
            pl.BlockSpec((_BLK, 1), lambda i: (i, 0)),
            pl.BlockSpec((1, D_OUT), lambda i: (0, 0)),
        ],
        out_specs=pl.BlockSpec((_BLK, D_OUT), lambda i: (i, 0)),
        out_shape=jax.ShapeDtypeStruct((NP, D_OUT), jnp.float32),
    )(t0, t1, u2, dis, b2)


def kernel(x, edge_index, edge_weight, W1, b1, W2, b2):
    src = edge_index[0]
    dst = edge_index[1]
    pad_e = EP - E
    srcp = jnp.concatenate([src, jnp.zeros((pad_e,), jnp.int32)])
    dstp = jnp.concatenate([dst, jnp.zeros((pad_e,), jnp.int32)])
    ewp = jnp.concatenate([edge_weight, jnp.zeros((pad_e,), jnp.float32)])
    xp = jnp.concatenate([x, jnp.zeros((NP - N, D_IN), jnp.float32)])
    zeros1 = jnp.zeros((NP,), jnp.float32)
    zeros2 = jnp.zeros((NP, D_IN), jnp.float32)

    degp = _deg_kernel(dstp, ewp, zeros1)
    dis, u1 = _tc1(degp[0].reshape(NP, 1), degp[1].reshape(NP, 1), xp)
    t1 = _agg_kernel(u1, srcp, dstp, ewp, zeros2)
    u2 = _tc2(t1[0], t1[1], u1, dis, W1, b1.reshape(1, D_H), W2)
    t2 = _agg_kernel(u2, srcp, dstp, ewp, zeros2)
    z = _tc3(t2[0], t2[1], u2, dis, b2.reshape(1, D_OUT))
    return z[:N]


# speedup vs baseline: 10.4061x; 1.7233x over previous
import functools

import jax
import jax.numpy as jnp
from jax import lax
from jax.experimental import pallas as pl
from jax.experimental.pallas import tpu as pltpu
from jax.experimental.pallas import tpu_sc as plsc

N = 10000
NP = 10240
E = 320000
D_IN = 128
D_H = 256
D_OUT = 128

NC, NS, LANES = 2, 16, 16
NW = NC * NS
CHUNK = 128
CHUNKS_PER_W = 80
EDGES_PER_W = CHUNK * CHUNKS_PER_W
EP = NW * EDGES_PER_W
NCHUNKS = EP // CHUNK
ROWS_PER_SUB = NP // NS
_GDN = lax.GatherDimensionNumbers(offset_dims=(), collapsed_slice_dims=(0,),
                                  start_index_map=(0,))


def _lane_broadcast(vec, l):
    idx = jnp.full((LANES, 1), l, jnp.int32)
    return lax.gather(vec, idx, _GDN, slice_sizes=(1,),
                      mode=lax.GatherScatterMode.PROMISE_IN_BOUNDS)

_MESH = plsc.VectorSubcoreMesh(core_axis_name="c", subcore_axis_name="s",
                               num_cores=NC, num_subcores=NS)
_SC_PARAMS = pltpu.CompilerParams(needs_layout_passes=False)


def _deg_body(dst_hbm, ew_hbm, zeros1_hbm, out_hbm, dstv, ewv, deg_sh, sem):
    c = lax.axis_index("c")
    s = lax.axis_index("s")
    wid = c * NS + s
    row0 = s * ROWS_PER_SUB
    pltpu.sync_copy(zeros1_hbm.at[pl.ds(row0, ROWS_PER_SUB)],
                    deg_sh.at[pl.ds(row0, ROWS_PER_SUB)])
    base = wid * CHUNKS_PER_W
    pltpu.sync_copy(dst_hbm.at[pl.ds(base, CHUNKS_PER_W)], dstv)
    pltpu.sync_copy(ew_hbm.at[pl.ds(base, CHUNKS_PER_W)], ewv)
    plsc.subcore_barrier()

    def chunk(j, carry):
        pltpu.async_copy(ewv.at[j], deg_sh.at[dstv.at[j]], sem, add=True)

        @pl.when(j >= 8)
        def _():
            pltpu.make_async_copy(ewv.at[0], deg_sh.at[dstv.at[0]],
                                  sem).wait()

        return carry

    lax.fori_loop(0, CHUNKS_PER_W, chunk, 0)

    def drain(j, carry):
        pltpu.make_async_copy(ewv.at[0], deg_sh.at[dstv.at[0]], sem).wait()
        return carry

    lax.fori_loop(0, 8, drain, 0)
    plsc.subcore_barrier()
    pltpu.sync_copy(deg_sh.at[pl.ds(row0, ROWS_PER_SUB)],
                    out_hbm.at[c, pl.ds(row0, ROWS_PER_SUB)])


_deg_kernel = functools.partial(
    pl.kernel,
    out_type=jax.ShapeDtypeStruct((NC, NP), jnp.float32),
    mesh=_MESH,
    compiler_params=_SC_PARAMS,
    scratch_types=[
        pltpu.VMEM((CHUNKS_PER_W, CHUNK), jnp.int32),
        pltpu.VMEM((CHUNKS_PER_W, CHUNK), jnp.float32),
        pltpu.MemorySpace.VMEM_SHARED((NP,), jnp.float32),
        pltpu.SemaphoreType.DMA,
    ],
)(_deg_body)


GRP = 40


def _agg_body(u_hbm, src_hbm, dst_hbm, ew_hbm, zeros2_hbm, out_hbm,
              srcv, dstv, ewv, gbuf0, gbuf1, t_sh, gsem0, gsem1):
    c = lax.axis_index("c")
    s = lax.axis_index("s")
    wid = c * NS + s
    row0 = s * ROWS_PER_SUB
    gbufs = (gbuf0, gbuf1)
    gsems = (gsem0, gsem1)

    pltpu.sync_copy(zeros2_hbm.at[pl.ds(row0, ROWS_PER_SUB)],
                    t_sh.at[pl.ds(row0, ROWS_PER_SUB)])
    plsc.subcore_barrier()

    for G in range(CHUNKS_PER_W // GRP):
        gbase = wid * CHUNKS_PER_W + G * GRP
        pltpu.sync_copy(src_hbm.at[pl.ds(gbase, GRP)], srcv)
        pltpu.sync_copy(dst_hbm.at[pl.ds(gbase, GRP)], dstv)
        pltpu.sync_copy(ew_hbm.at[pl.ds(gbase, GRP)], ewv)
        for b in range(2):
            pltpu.async_copy(u_hbm.at[srcv.at[b]], gbufs[b], gsems[b])

        def pair(p, carry):
            for b in range(2):
                jj = 2 * p + b
                pltpu.make_async_copy(u_hbm.at[srcv.at[jj]], gbufs[b],
                                      gsems[b]).wait()

                def grp_fn(g, c2):
                    ewg = ewv[jj, pl.ds(g * LANES, LANES)]
                    for l in range(LANES):
                        bew = _lane_broadcast(ewg, l)
                        e = g * LANES + l
                        for f in range(D_IN // LANES):
                            sl = pl.ds(f * LANES, LANES)
                            gbufs[b][e, sl] = gbufs[b][e, sl] * bew
                    return c2

                lax.fori_loop(0, CHUNK // LANES, grp_fn, 0)
                pltpu.sync_copy(gbufs[b], t_sh.at[dstv.at[jj]], add=True)

                @pl.when(jj + 2 < GRP)
                def _():
                    pltpu.async_copy(u_hbm.at[srcv.at[jj + 2]], gbufs[b],
                                     gsems[b])

            return carry

        lax.fori_loop(0, GRP // 2, pair, 0)

    plsc.subcore_barrier()
    pltpu.sync_copy(t_sh.at[pl.ds(row0, ROWS_PER_SUB)],
                    out_hbm.at[c, pl.ds(row0, ROWS_PER_SUB)])


_agg_kernel = functools.partial(
    pl.kernel,
    out_type=jax.ShapeDtypeStruct((NC, NP, D_IN), jnp.float32),
    mesh=_MESH,
    compiler_params=_SC_PARAMS,
    scratch_types=[
        pltpu.VMEM((GRP, CHUNK), jnp.int32),
        pltpu.VMEM((GRP, CHUNK), jnp.int32),
        pltpu.VMEM((GRP, CHUNK), jnp.float32),
        pltpu.VMEM((CHUNK, D_IN), jnp.float32),
        pltpu.VMEM((CHUNK, D_IN), jnp.float32),
        pltpu.MemorySpace.VMEM_SHARED((NP, D_IN), jnp.float32),
        pltpu.SemaphoreType.DMA,
        pltpu.SemaphoreType.DMA,
    ],
)(_agg_body)


_BLK = 1024
_GRID = NP // _BLK


def _tc1_body(d0_ref, d1_ref, x_ref, dis_ref, u1_ref):
    deg = d0_ref[...] + d1_ref[...] + 1.0
    dis = 1.0 / jnp.sqrt(deg)
    dis_ref[...] = dis
    u1_ref[...] = x_ref[...] * dis


def _tc1(d0, d1, xp):
    return pl.pallas_call(
        _tc1_body,
        grid=(_GRID,),
        in_specs=[
            pl.BlockSpec((_BLK, 1), lambda i: (i, 0)),
            pl.BlockSpec((_BLK, 1), lambda i: (i, 0)),
            pl.BlockSpec((_BLK, D_IN), lambda i: (i, 0)),
        ],
        out_specs=[
            pl.BlockSpec((_BLK, 1), lambda i: (i, 0)),
            pl.BlockSpec((_BLK, D_IN), lambda i: (i, 0)),
        ],
        out_shape=[
            jax.ShapeDtypeStruct((NP, 1), jnp.float32),
            jax.ShapeDtypeStruct((NP, D_IN), jnp.float32),
        ],
    )(d0, d1, xp)


def _tc2_body(t0_ref, t1_ref, u1_ref, dis_ref, w1_ref, b1_ref, w2_ref,
              u2_ref):
    g1 = (t0_ref[...] + t1_ref[...] + u1_ref[...]) * dis_ref[...]
    h = jnp.dot(g1, w1_ref[...], preferred_element_type=jnp.float32,
                precision=lax.Precision.HIGHEST) + b1_ref[...]
    h = jnp.maximum(h, 0.0)
    u2 = jnp.dot(h, w2_ref[...], preferred_element_type=jnp.float32,
                 precision=lax.Precision.HIGHEST)
    u2_ref[...] = u2 * dis_ref[...]


def _tc2(t0, t1, u1, dis, W1, b1, W2):
    return pl.pallas_call(
        _tc2_body,
        grid=(_GRID,),
        in_specs=[
            pl.BlockSpec((_BLK, D_IN), lambda i: (i, 0)),
            pl.BlockSpec((_BLK, D_IN), lambda i: (i, 0)),
            pl.BlockSpec((_BLK, D_IN), lambda i: (i, 0)),
            pl.BlockSpec((_BLK, 1), lambda i: (i, 0)),
            pl.BlockSpec((D_IN, D_H), lambda i: (0, 0)),
            pl.BlockSpec((1, D_H), lambda i: (0, 0)),
            pl.BlockSpec((D_H, D_OUT), lambda i: (0, 0)),
        ],
        out_specs=pl.BlockSpec((_BLK, D_OUT), lambda i: (i, 0)),
        out_shape=jax.ShapeDtypeStruct((NP, D_OUT), jnp.float32),
    )(t0, t1, u1, dis, W1, b1, W2)


def _tc3_body(t0_ref, t1_ref, u2_ref, dis_ref, b2_ref, z_ref):
    z_ref[...] = ((t0_ref[...] + t1_ref[...] + u2_ref[...]) * dis_ref[...]
                  + b2_ref[...])


def _tc3(t0, t1, u2, dis, b2):
    return pl.pallas_call(
        _tc3_body,
        grid=(_GRID,),
        in_specs=[
            pl.BlockSpec((_BLK, D_OUT), lambda i: (i, 0)),
            pl.BlockSpec((_BLK, D_OUT), lambda i: (i, 0)),
            pl.BlockSpec((_BLK, D_OUT), lambda i: (i, 0)),
            pl.BlockSpec((_BLK, 1), lambda i: (i, 0)),
            pl.BlockSpec((1, D_OUT), lambda i: (0, 0)),
        ],
        out_specs=pl.BlockSpec((_BLK, D_OUT), lambda i: (i, 0)),
        out_shape=jax.ShapeDtypeStruct((NP, D_OUT), jnp.float32),
    )(t0, t1, u2, dis, b2)


def kernel(x, edge_index, edge_weight, W1, b1, W2, b2):
    src = edge_index[0]
    dst = edge_index[1]
    pad_e = EP - E
    srcp = jnp.concatenate([src, jnp.zeros((pad_e,), jnp.int32)])
    dstp = jnp.concatenate([dst, jnp.zeros((pad_e,), jnp.int32)])
    ewp = jnp.concatenate([edge_weight, jnp.zeros((pad_e,), jnp.float32)])
    src2 = srcp.reshape(NCHUNKS, CHUNK)
    dst2 = dstp.reshape(NCHUNKS, CHUNK)
    ew2 = ewp.reshape(NCHUNKS, CHUNK)
    xp = jnp.concatenate([x, jnp.zeros((NP - N, D_IN), jnp.float32)])
    zeros1 = jnp.zeros((NP,), jnp.float32)
    zeros2 = jnp.zeros((NP, D_IN), jnp.float32)

    degp = _deg_kernel(dst2, ew2, zeros1)
    dis, u1 = _tc1(degp[0].reshape(NP, 1), degp[1].reshape(NP, 1), xp)
    t1 = _agg_kernel(u1, src2, dst2, ew2, zeros2)
    u2 = _tc2(t1[0], t1[1], u1, dis, W1, b1.reshape(1, D_H), W2)
    t2 = _agg_kernel(u2, src2, dst2, ew2, zeros2)
    z = _tc3(t2[0], t2[1], u2, dis, b2.reshape(1, D_OUT))
    return z[:N]


# speedup vs baseline: 14.8008x; 1.4223x over previous
import functools

import jax
import jax.numpy as jnp
from jax import lax
from jax.experimental import pallas as pl
from jax.experimental.pallas import tpu as pltpu
from jax.experimental.pallas import tpu_sc as plsc

N = 10000
NP = 10240
E = 320000
D_IN = 128
D_H = 256
D_OUT = 128
DH2 = D_IN // 2

NC, NS, LANES = 2, 16, 16
NW = NC * NS
CHUNK = 128
CHUNKS_PER_W = 80
EDGES_PER_W = CHUNK * CHUNKS_PER_W
EP = NW * EDGES_PER_W
NCHUNKS = EP // CHUNK
CHUNKS_PER_T = NCHUNKS // NS
GRP = 80
ROWS_PER_SUB = NP // NS

_MESH = plsc.VectorSubcoreMesh(core_axis_name="c", subcore_axis_name="s",
                               num_cores=NC, num_subcores=NS)
_SC_PARAMS = pltpu.CompilerParams(needs_layout_passes=False,
                                  use_tc_tiling_on_sc=False)

_GDN = lax.GatherDimensionNumbers(offset_dims=(), collapsed_slice_dims=(0,),
                                  start_index_map=(0,))


def _lane_broadcast(vec, l):
    idx = jnp.full((LANES, 1), l, jnp.int32)
    return lax.gather(vec, idx, _GDN, slice_sizes=(1,),
                      mode=lax.GatherScatterMode.PROMISE_IN_BOUNDS)


def _deg_body(dst_hbm, ew_hbm, zeros1_hbm, out_hbm, dstv, ewv, deg_sh, sem):
    c = lax.axis_index("c")
    s = lax.axis_index("s")
    wid = c * NS + s
    row0 = s * ROWS_PER_SUB
    pltpu.sync_copy(zeros1_hbm.at[pl.ds(row0, ROWS_PER_SUB)],
                    deg_sh.at[pl.ds(row0, ROWS_PER_SUB)])
    base = wid * CHUNKS_PER_W
    pltpu.sync_copy(dst_hbm.at[pl.ds(base, CHUNKS_PER_W)], dstv)
    pltpu.sync_copy(ew_hbm.at[pl.ds(base, CHUNKS_PER_W)], ewv)
    plsc.subcore_barrier()

    def chunk(j, carry):
        pltpu.async_copy(ewv.at[j], deg_sh.at[dstv.at[j]], sem, add=True)

        @pl.when(j >= 8)
        def _():
            pltpu.make_async_copy(ewv.at[0], deg_sh.at[dstv.at[0]],
                                  sem).wait()

        return carry

    lax.fori_loop(0, CHUNKS_PER_W, chunk, 0)

    def drain(j, carry):
        pltpu.make_async_copy(ewv.at[0], deg_sh.at[dstv.at[0]], sem).wait()
        return carry

    lax.fori_loop(0, 8, drain, 0)
    plsc.subcore_barrier()
    pltpu.sync_copy(deg_sh.at[pl.ds(row0, ROWS_PER_SUB)],
                    out_hbm.at[c, pl.ds(row0, ROWS_PER_SUB)])


_deg_kernel = functools.partial(
    pl.kernel,
    out_type=jax.ShapeDtypeStruct((NC, NP), jnp.float32),
    mesh=_MESH,
    compiler_params=_SC_PARAMS,
    scratch_types=[
        pltpu.VMEM((CHUNKS_PER_W, CHUNK), jnp.int32),
        pltpu.VMEM((CHUNKS_PER_W, CHUNK), jnp.float32),
        pltpu.MemorySpace.VMEM_SHARED((NP,), jnp.float32),
        pltpu.SemaphoreType.DMA,
    ],
)(_deg_body)


def _agg_body(u_hbm, src_hbm, dst_hbm, ew_hbm, zeros2_hbm, out_hbm,
              srcv, dstv, ewv, gbuf0, gbuf1, sbuf0, sbuf1, t_sh,
              gsem0, gsem1, ssem0, ssem1):
    c = lax.axis_index("c")
    s = lax.axis_index("s")
    row0 = s * ROWS_PER_SUB
    gbufs = (gbuf0, gbuf1)
    sbufs = (sbuf0, sbuf1)
    gsems = (gsem0, gsem1)
    ssems = (ssem0, ssem1)

    pltpu.sync_copy(zeros2_hbm.at[pl.ds(row0, ROWS_PER_SUB)],
                    t_sh.at[pl.ds(row0, ROWS_PER_SUB)])
    plsc.subcore_barrier()
    uoff = jnp.zeros((LANES,), jnp.int32) + c * NP

    for G in range(CHUNKS_PER_T // GRP):
        gbase = s * CHUNKS_PER_T + G * GRP
        pltpu.sync_copy(src_hbm.at[pl.ds(gbase, GRP)], srcv)
        pltpu.sync_copy(dst_hbm.at[pl.ds(gbase, GRP)], dstv)
        pltpu.sync_copy(ew_hbm.at[pl.ds(gbase, GRP)], ewv)

        def offs(r, carry):
            for f in range(CHUNK // LANES):
                sl = pl.ds(f * LANES, LANES)
                srcv[r, sl] = srcv[r, sl] + uoff
            return carry

        lax.fori_loop(0, GRP, offs, 0)

        for b in range(2):
            pltpu.async_copy(u_hbm.at[srcv.at[b]], gbufs[b], gsems[b])

        def pair(p, carry):
            for b in range(2):
                jj = 2 * p + b
                pltpu.make_async_copy(u_hbm.at[srcv.at[jj]], gbufs[b],
                                      gsems[b]).wait()

                if G == 0:
                    @pl.when(p > 0)
                    def _():
                        pltpu.make_async_copy(sbufs[b], t_sh.at[dstv.at[jj]],
                                              ssems[b]).wait()
                else:
                    pltpu.make_async_copy(sbufs[b], t_sh.at[dstv.at[jj]],
                                          ssems[b]).wait()

                def grp_fn(g, c2):
                    ewg = ewv[jj, pl.ds(g * LANES, LANES)]
                    for l in range(LANES):
                        bew = _lane_broadcast(ewg, l)
                        e = g * LANES + l
                        for f in range(DH2 // LANES):
                            sl = pl.ds(f * LANES, LANES)
                            sbufs[b][e, sl] = gbufs[b][e, sl] * bew
                    return c2

                lax.fori_loop(0, CHUNK // LANES, grp_fn, 0)

                @pl.when(jj + 2 < GRP)
                def _():
                    pltpu.async_copy(u_hbm.at[srcv.at[jj + 2]], gbufs[b],
                                     gsems[b])

                pltpu.async_copy(sbufs[b], t_sh.at[dstv.at[jj]], ssems[b],
                                 add=True)

            return carry

        lax.fori_loop(0, GRP // 2, pair, 0)

    for b in range(2):
        pltpu.make_async_copy(sbufs[b], t_sh.at[dstv.at[0]], ssems[b]).wait()
    plsc.subcore_barrier()
    pltpu.sync_copy(t_sh.at[pl.ds(row0, ROWS_PER_SUB)],
                    out_hbm.at[c, pl.ds(row0, ROWS_PER_SUB)])


_agg_kernel = functools.partial(
    pl.kernel,
    out_type=jax.ShapeDtypeStruct((NC, NP, DH2), jnp.float32),
    mesh=_MESH,
    compiler_params=_SC_PARAMS,
    scratch_types=[
        pltpu.VMEM((GRP, CHUNK), jnp.int32),
        pltpu.VMEM((GRP, CHUNK), jnp.int32),
        pltpu.VMEM((GRP, CHUNK), jnp.float32),
        pltpu.VMEM((CHUNK, DH2), jnp.float32),
        pltpu.VMEM((CHUNK, DH2), jnp.float32),
        pltpu.VMEM((CHUNK, DH2), jnp.float32),
        pltpu.VMEM((CHUNK, DH2), jnp.float32),
        pltpu.MemorySpace.VMEM_SHARED((NP, DH2), jnp.float32),
        pltpu.SemaphoreType.DMA,
        pltpu.SemaphoreType.DMA,
        pltpu.SemaphoreType.DMA,
        pltpu.SemaphoreType.DMA,
    ],
)(_agg_body)


_BLK = 1024
_GRID = NP // _BLK


def _tc1_body(d0_ref, d1_ref, x_ref, dis_ref, u1_ref):
    deg = d0_ref[...] + d1_ref[...] + 1.0
    dis = 1.0 / jnp.sqrt(deg)
    dis_ref[...] = dis
    u1 = x_ref[...] * dis
    u1_ref[0] = u1[:, :DH2]
    u1_ref[1] = u1[:, DH2:]


def _tc1(d0, d1, xp):
    return pl.pallas_call(
        _tc1_body,
        grid=(_GRID,),
        in_specs=[
            pl.BlockSpec((_BLK, 1), lambda i: (i, 0)),
            pl.BlockSpec((_BLK, 1), lambda i: (i, 0)),
            pl.BlockSpec((_BLK, D_IN), lambda i: (i, 0)),
        ],
        out_specs=[
            pl.BlockSpec((_BLK, 1), lambda i: (i, 0)),
            pl.BlockSpec((2, _BLK, DH2), lambda i: (0, i, 0)),
        ],
        out_shape=[
            jax.ShapeDtypeStruct((NP, 1), jnp.float32),
            jax.ShapeDtypeStruct((2, NP, DH2), jnp.float32),
        ],
    )(d0, d1, xp)


def _tc2_body(t_ref, u1_ref, dis_ref, w1_ref, b1_ref, w2_ref, u2_ref):
    dis = dis_ref[...]
    g1 = jnp.concatenate(
        [(t_ref[0] + u1_ref[0]) * dis, (t_ref[1] + u1_ref[1]) * dis], axis=1)
    h = jnp.dot(g1, w1_ref[...], preferred_element_type=jnp.float32,
                precision=lax.Precision.HIGHEST) + b1_ref[...]
    h = jnp.maximum(h, 0.0)
    u2 = jnp.dot(h, w2_ref[...], preferred_element_type=jnp.float32,
                 precision=lax.Precision.HIGHEST)
    u2 = u2 * dis
    u2_ref[0] = u2[:, :DH2]
    u2_ref[1] = u2[:, DH2:]


def _tc2(t1, u1, dis, W1, b1, W2):
    return pl.pallas_call(
        _tc2_body,
        grid=(_GRID,),
        in_specs=[
            pl.BlockSpec((2, _BLK, DH2), lambda i: (0, i, 0)),
            pl.BlockSpec((2, _BLK, DH2), lambda i: (0, i, 0)),
            pl.BlockSpec((_BLK, 1), lambda i: (i, 0)),
            pl.BlockSpec((D_IN, D_H), lambda i: (0, 0)),
            pl.BlockSpec((1, D_H), lambda i: (0, 0)),
            pl.BlockSpec((D_H, D_OUT), lambda i: (0, 0)),
        ],
        out_specs=pl.BlockSpec((2, _BLK, DH2), lambda i: (0, i, 0)),
        out_shape=jax.ShapeDtypeStruct((2, NP, DH2), jnp.float32),
    )(t1, u1, dis, W1, b1, W2)


def _tc3_body(t_ref, u2_ref, dis_ref, b2_ref, z_ref):
    dis = dis_ref[...]
    z = jnp.concatenate(
        [(t_ref[0] + u2_ref[0]) * dis, (t_ref[1] + u2_ref[1]) * dis], axis=1)
    z_ref[...] = z + b2_ref[...]


def _tc3(t2, u2, dis, b2):
    return pl.pallas_call(
        _tc3_body,
        grid=(_GRID,),
        in_specs=[
            pl.BlockSpec((2, _BLK, DH2), lambda i: (0, i, 0)),
            pl.BlockSpec((2, _BLK, DH2), lambda i: (0, i, 0)),
            pl.BlockSpec((_BLK, 1), lambda i: (i, 0)),
            pl.BlockSpec((1, D_OUT), lambda i: (0, 0)),
        ],
        out_specs=pl.BlockSpec((_BLK, D_OUT), lambda i: (i, 0)),
        out_shape=jax.ShapeDtypeStruct((NP, D_OUT), jnp.float32),
    )(t2, u2, dis, b2)


def kernel(x, edge_index, edge_weight, W1, b1, W2, b2):
    src = edge_index[0]
    dst = edge_index[1]
    pad_e = EP - E
    srcp = jnp.concatenate([src, jnp.zeros((pad_e,), jnp.int32)])
    dstp = jnp.concatenate([dst, jnp.zeros((pad_e,), jnp.int32)])
    ewp = jnp.concatenate([edge_weight, jnp.zeros((pad_e,), jnp.float32)])
    src2 = srcp.reshape(NCHUNKS, CHUNK)
    dst2 = dstp.reshape(NCHUNKS, CHUNK)
    ew2 = ewp.reshape(NCHUNKS, CHUNK)
    xp = jnp.concatenate([x, jnp.zeros((NP - N, D_IN), jnp.float32)])
    zeros1 = jnp.zeros((NP,), jnp.float32)
    zeros2 = jnp.zeros((NP, DH2), jnp.float32)

    degp = _deg_kernel(dst2, ew2, zeros1)
    dis, u1s = _tc1(degp[0].reshape(NP, 1), degp[1].reshape(NP, 1), xp)
    t1 = _agg_kernel(u1s.reshape(2 * NP, DH2), src2, dst2, ew2, zeros2)
    u2s = _tc2(t1, u1s, dis, W1, b1.reshape(1, D_H), W2)
    t2 = _agg_kernel(u2s.reshape(2 * NP, DH2), src2, dst2, ew2, zeros2)
    z = _tc3(t2, u2s, dis, b2.reshape(1, D_OUT))
    return z[:N]
